# baseline (speedup 1.0000x reference)
import functools

import jax
import jax.numpy as jnp
from jax import lax
from jax.experimental import pallas as pl
from jax.experimental.pallas import tpu as pltpu
from jax.experimental.pallas import tpu_sc as plsc

_NT = 50


def _sc_call(xt):
    info = plsc.get_sparse_core_info()
    nc = info.num_cores
    mesh = plsc.VectorSubcoreMesh(core_axis_name="c", subcore_axis_name="s")

    @functools.partial(
        pl.kernel,
        mesh=mesh,
        out_type=jax.ShapeDtypeStruct((100, 64, 4096), jnp.float32),
        scratch_types=[
            pltpu.VMEM((8, 4096), jnp.float32),
            pltpu.VMEM((4, 4096), jnp.float32),
            pltpu.VMEM((4, 4096), jnp.float32),
            pltpu.VMEM((4, 4096), jnp.float32),
            pltpu.VMEM((4, 4096), jnp.float32),
            pltpu.SemaphoreType.DMA,
            pltpu.SemaphoreType.DMA,
            pltpu.SemaphoreType.DMA,
            pltpu.SemaphoreType.DMA,
            pltpu.SemaphoreType.DMA,
            pltpu.SemaphoreType.DMA,
            pltpu.SemaphoreType.DMA,
            pltpu.SemaphoreType.DMA,
        ],
    )
    def sc(xt_hbm, out_hbm, off_v, b0, b1, b2, b3, i0, i1, i2, i3, o0, o1, o2, o3):
        wid = lax.axis_index("s") * nc + lax.axis_index("c")
        jj8 = (wid % 8) * 8
        krem = wid // 8

        pltpu.sync_copy(xt_hbm.at[0, pl.ds(jj8, 8), :], off_v)

        bufs = (b0, b1, b2, b3)
        sin = (i0, i1, i2, i3)
        sout = (o0, o1, o2, o3)

        def _make_subtract(w_v, h):
            def subtract(i, carry):
                r = i >> 6
                cb = (i & 63) * 64
                for u in range(4):
                    w_v[r, pl.ds(cb + u * 16, 16)] = (
                        w_v[r, pl.ds(cb + u * 16, 16)]
                        - off_v[h * 4 + r, pl.ds(cb + u * 16, 16)]
                    )
                return carry

            return subtract

        subs = tuple(
            tuple(_make_subtract(b, h) for h in range(2)) for b in bufs
        )

        def start_in(t):
            k = krem + 4 * (t // 2)
            return pltpu.async_copy(
                xt_hbm.at[50 + k, pl.ds(jj8 + (t % 2) * 4, 4), :],
                bufs[t % 4],
                sin[t % 4],
            )

        def start_out(t):
            k = krem + 4 * (t // 2)
            return pltpu.async_copy(
                bufs[t % 4],
                out_hbm.at[k, pl.ds(jj8 + (t % 2) * 4, 4), :],
                sout[t % 4],
            )

        in_h = {0: start_in(0), 1: start_in(1)}
        out_h = {}
        for t in range(_NT):
            if t + 2 < _NT:
                if t >= 2:
                    out_h.pop(t - 2).wait()
                in_h[t + 2] = start_in(t + 2)
            in_h.pop(t).wait()
            lax.fori_loop(0, 256, subs[t % 4][t % 2], 0)
            out_h[t] = start_out(t)
        out_h.pop(_NT - 2).wait()
        out_h.pop(_NT - 1).wait()

    return sc(xt)


def kernel(x, W):
    del W
    xt = jnp.transpose(x, (1, 2, 0))
    out_t = _sc_call(xt)
    return jnp.transpose(out_t, (2, 0, 1))


# speedup vs baseline: 4.8149x; 4.8149x over previous
import jax
import jax.numpy as jnp
from jax.experimental import pallas as pl

_G = 10


def _body(off_ref, x_ref, o_ref):
    o_ref[...] = x_ref[...] - off_ref[...]


def kernel(x, W):
    del W
    n, dim, d = x.shape
    xt = jnp.transpose(x, (1, 2, 0))

    out_t = pl.pallas_call(
        _body,
        grid=(100 // _G,),
        in_specs=[
            pl.BlockSpec((1, d, n), lambda j: (0, 0, 0)),
            pl.BlockSpec((_G, d, n), lambda j: (j + 50 // _G, 0, 0)),
        ],
        out_specs=pl.BlockSpec((_G, d, n), lambda j: (j, 0, 0)),
        out_shape=jax.ShapeDtypeStruct((100, d, n), x.dtype),
    )(xt, xt)
    return jnp.transpose(out_t, (2, 0, 1))
